# baseline (speedup 1.0000x reference)
import functools

import jax
import jax.numpy as jnp
from jax import lax
from jax.experimental import pallas as pl
from jax.experimental.pallas import tpu as pltpu
from jax.experimental.pallas import tpu_sc as plsc

N = 10000
F = 128
E = 320000
HID = 128
NC, NS = 2, 16
CHUNK = 128
CHUNKS = 160
EPAD = NS * CHUNKS * CHUNK
NPAD = 10112
ZPT = NPAD // NS
DUMMY = N
NBUF = 2


def _sc_hops():
    mesh = plsc.VectorSubcoreMesh(core_axis_name="c", subcore_axis_name="s")
    out_sd = jax.ShapeDtypeStruct((NC * NPAD, F), jnp.float32)

    @functools.partial(
        pl.kernel,
        out_type=(out_sd, out_sd, out_sd),
        mesh=mesh,
        scratch_types=[
            pltpu.VMEM((CHUNKS // 2, CHUNK), jnp.int32),
            pltpu.VMEM((NBUF, CHUNK), jnp.int32),
        ] + [pltpu.VMEM((CHUNK, F), jnp.float32)] * NBUF
          + [pltpu.VMEM_SHARED((NPAD, F), jnp.float32)]
          + [pltpu.SemaphoreType.DMA] * NBUF,
    )
    def k(xs, srcb, dstb, zrow, m1, m2, m3, srcv, dstr, *rest):
        bufs = rest[:NBUF]
        acc = rest[NBUF]
        sems = rest[NBUF + 1:]
        c = lax.axis_index("c")
        s = lax.axis_index("s")
        HC = CHUNKS // 2

        table = xs
        for m in (m1, m2, m3):
            pltpu.sync_copy(zrow.at[pl.ds(s * ZPT, ZPT)],
                            acc.at[pl.ds(s * ZPT, ZPT)])
            plsc.subcore_barrier()

            for p in range(2):
                pltpu.sync_copy(srcb.at[c, s, pl.ds(p * HC, HC)], srcv)

                def fire(chunk, b, table=table, p=p):
                    pltpu.async_copy(table.at[srcv.at[chunk]], bufs[b],
                                     sems[b])
                    pltpu.async_copy(dstb.at[s, p * HC + chunk], dstr.at[b],
                                     sems[b])

                def drain(chunk, b, table=table, p=p):
                    pltpu.make_async_copy(table.at[srcv.at[chunk]], bufs[b],
                                          sems[b]).wait()
                    pltpu.make_async_copy(dstb.at[s, p * HC + chunk],
                                          dstr.at[b], sems[b]).wait()

                for b in range(NBUF):
                    fire(b, b)

                @pl.loop(0, HC, step=NBUF)
                def _(j):
                    for b in range(NBUF):
                        cur = j + b
                        drain(cur, b)
                        pltpu.sync_copy(bufs[b], acc.at[dstr.at[b]], add=True)

                        @pl.when(cur + NBUF < HC)
                        def _():
                            fire(cur + NBUF, b)

            plsc.subcore_barrier()
            pltpu.sync_copy(acc.at[pl.ds(s * ZPT, ZPT)],
                            m.at[pl.ds(c * NPAD + s * ZPT, ZPT)])
            plsc.subcore_barrier()
            table = m

    return k


def _gru_block(x_ref, m3_ref, m2_ref, m1_ref, wih_ref, whh_ref, bih_ref,
               bhh_ref, o_ref):
    wih = wih_ref[...]
    whh = whh_ref[...]
    bih = bih_ref[...]
    bhh = bhh_ref[...]

    seq = (m3_ref[...], m2_ref[...], m1_ref[...], x_ref[...])
    h = jnp.zeros((x_ref.shape[0], HID), jnp.float32)
    for xt in seq:
        gi = lax.dot_general(xt, wih, (((1,), (1,)), ((), ())),
                             preferred_element_type=jnp.float32) + bih
        gh = lax.dot_general(h, whh, (((1,), (1,)), ((), ())),
                             preferred_element_type=jnp.float32) + bhh
        r = jax.nn.sigmoid(gi[:, :HID] + gh[:, :HID])
        z = jax.nn.sigmoid(gi[:, HID:2 * HID] + gh[:, HID:2 * HID])
        n = jnp.tanh(gi[:, 2 * HID:] + r * gh[:, 2 * HID:])
        h = (1.0 - z) * n + z * h
    o_ref[...] = h


def _gru(x, m1, m2, m3, W_ih, W_hh, b_ih, b_hh, blk=1000):
    nblk = N // blk
    mspec = pl.BlockSpec((blk, F), lambda b: (b, 0))
    wspec = pl.BlockSpec((3 * HID, F), lambda b: (0, 0))
    bspec = pl.BlockSpec((1, 3 * HID), lambda b: (0, 0))
    return pl.pallas_call(
        _gru_block,
        grid=(nblk,),
        in_specs=[mspec, mspec, mspec, mspec, wspec, wspec, bspec, bspec],
        out_specs=pl.BlockSpec((blk, HID), lambda b: (b, 0)),
        out_shape=jax.ShapeDtypeStruct((N, HID), jnp.float32),
    )(x, m3, m2, m1, W_ih, W_hh, b_ih.reshape(1, -1), b_hh.reshape(1, -1))


def kernel(x, ei, W_ih, W_hh, b_ih, b_hh):
    x = x.astype(jnp.float32)
    ei = ei.astype(jnp.int32)
    xp = jnp.pad(x, ((0, NPAD - N), (0, 0)))
    xs = jnp.concatenate([xp, xp], axis=0)
    src = jnp.pad(ei[0], (0, EPAD - E)).reshape(NS, CHUNKS, CHUNK)
    srcb = src[None] + (jnp.arange(NC, dtype=jnp.int32)
                        * NPAD)[:, None, None, None]
    dstb = jnp.pad(ei[1], (0, EPAD - E),
                   constant_values=DUMMY).reshape(NS, CHUNKS, CHUNK)
    zrow = jnp.zeros((NPAD, F), jnp.float32)
    m1, m2, m3 = _sc_hops()(xs, srcb, dstb, zrow)
    return _gru(x, m1[:N], m2[:N], m3[:N], W_ih, W_hh, b_ih, b_hh)


# speedup vs baseline: 1.3207x; 1.3207x over previous
import functools

import jax
import jax.numpy as jnp
from jax import lax
from jax.experimental import pallas as pl
from jax.experimental.pallas import tpu as pltpu
from jax.experimental.pallas import tpu_sc as plsc

N = 10000
F = 128
E = 320000
HID = 128
NC, NS = 2, 16
CHUNK = 80
CHUNKS = 128
EPAD = NC * NS * CHUNKS * CHUNK
NPAD = 10112
ZPT = NPAD // NS
DUMMY = N
NBUF = 4


def _sc_hop():
    mesh = plsc.VectorSubcoreMesh(core_axis_name="c", subcore_axis_name="s")
    out_sd = jax.ShapeDtypeStruct((NC, NPAD, F), jnp.float32)

    @functools.partial(
        pl.kernel,
        out_type=out_sd,
        mesh=mesh,
        scratch_types=[
            pltpu.VMEM((CHUNKS // 2, CHUNK), jnp.int32),
            pltpu.VMEM((NBUF, CHUNK), jnp.int32),
        ] + [pltpu.VMEM((CHUNK, F), jnp.float32)] * NBUF
          + [pltpu.VMEM_SHARED((NPAD, F), jnp.float32)]
          + [pltpu.SemaphoreType.DMA] * NBUF,
    )
    def k(table, srcb, dstb, zrow, m, srcv, dstr, *rest):
        bufs = rest[:NBUF]
        acc = rest[NBUF]
        sems = rest[NBUF + 1:]
        c = lax.axis_index("c")
        s = lax.axis_index("s")
        HC = CHUNKS // 2
        pltpu.sync_copy(zrow.at[pl.ds(s * ZPT, ZPT)],
                        acc.at[pl.ds(s * ZPT, ZPT)])
        plsc.subcore_barrier()

        for p in range(2):
            pltpu.sync_copy(srcb.at[c, s, pl.ds(p * HC, HC)], srcv)

            def fire(chunk, b, p=p):
                pltpu.async_copy(table.at[srcv.at[chunk]], bufs[b], sems[b])
                pltpu.async_copy(dstb.at[c, s, p * HC + chunk], dstr.at[b],
                                 sems[b])

            def drain(chunk, b, p=p):
                pltpu.make_async_copy(table.at[srcv.at[chunk]], bufs[b],
                                      sems[b]).wait()
                pltpu.make_async_copy(dstb.at[c, s, p * HC + chunk],
                                      dstr.at[b], sems[b]).wait()

            for b in range(NBUF):
                fire(b, b)

            @pl.loop(0, HC, step=NBUF)
            def _(j):
                for b in range(NBUF):
                    cur = j + b
                    drain(cur, b)
                    pltpu.sync_copy(bufs[b], acc.at[dstr.at[b]], add=True)

                    @pl.when(cur + NBUF < HC)
                    def _():
                        fire(cur + NBUF, b)

        plsc.subcore_barrier()
        pltpu.sync_copy(acc.at[pl.ds(s * ZPT, ZPT)],
                        m.at[c, pl.ds(s * ZPT, ZPT)])

    return k


def _add_block(a_ref, o_ref):
    o_ref[...] = a_ref[0] + a_ref[1]


def _combine(partial, blk=1264):
    return pl.pallas_call(
        _add_block,
        grid=(NPAD // blk,),
        in_specs=[pl.BlockSpec((NC, blk, F), lambda b: (0, b, 0))],
        out_specs=pl.BlockSpec((blk, F), lambda b: (b, 0)),
        out_shape=jax.ShapeDtypeStruct((NPAD, F), jnp.float32),
    )(partial)


def _gru_block(x_ref, m3_ref, m2_ref, m1_ref, wih_ref, whh_ref, bih_ref,
               bhh_ref, o_ref):
    wih = wih_ref[...]
    whh = whh_ref[...]
    bih = bih_ref[...]
    bhh = bhh_ref[...]

    m3b = m3_ref[...]
    seq = (m3b[0] + m3b[1], m2_ref[...], m1_ref[...], x_ref[...])
    h = jnp.zeros((x_ref.shape[0], HID), jnp.float32)
    for xt in seq:
        gi = lax.dot_general(xt, wih, (((1,), (1,)), ((), ())),
                             preferred_element_type=jnp.float32) + bih
        gh = lax.dot_general(h, whh, (((1,), (1,)), ((), ())),
                             preferred_element_type=jnp.float32) + bhh
        r = jax.nn.sigmoid(gi[:, :HID] + gh[:, :HID])
        z = jax.nn.sigmoid(gi[:, HID:2 * HID] + gh[:, HID:2 * HID])
        n = jnp.tanh(gi[:, 2 * HID:] + r * gh[:, 2 * HID:])
        h = (1.0 - z) * n + z * h
    o_ref[...] = h


def _gru(x, m1, m2, p3, W_ih, W_hh, b_ih, b_hh, blk=1000):
    nblk = N // blk
    mspec = pl.BlockSpec((blk, F), lambda b: (b, 0))
    wspec = pl.BlockSpec((3 * HID, F), lambda b: (0, 0))
    bspec = pl.BlockSpec((1, 3 * HID), lambda b: (0, 0))
    return pl.pallas_call(
        _gru_block,
        grid=(nblk,),
        in_specs=[
            mspec,
            pl.BlockSpec((NC, blk, F), lambda b: (0, b, 0)),
            mspec, mspec, wspec, wspec, bspec, bspec,
        ],
        out_specs=pl.BlockSpec((blk, HID), lambda b: (b, 0)),
        out_shape=jax.ShapeDtypeStruct((N, HID), jnp.float32),
    )(x, p3, m2, m1, W_ih, W_hh, b_ih.reshape(1, -1), b_hh.reshape(1, -1))


def kernel(x, ei, W_ih, W_hh, b_ih, b_hh):
    x = x.astype(jnp.float32)
    ei = ei.astype(jnp.int32)
    xp = jnp.pad(x, ((0, NPAD - N), (0, 0)))
    srcb = jnp.pad(ei[0], (0, EPAD - E)).reshape(NC, NS, CHUNKS, CHUNK)
    dstb = jnp.pad(ei[1], (0, EPAD - E),
                   constant_values=DUMMY).reshape(NC, NS, CHUNKS, CHUNK)
    zrow = jnp.zeros((NPAD, F), jnp.float32)
    hop = _sc_hop()
    p1 = hop(xp, srcb, dstb, zrow)
    m1 = _combine(p1)
    p2 = hop(m1, srcb, dstb, zrow)
    m2 = _combine(p2)
    p3 = hop(m2, srcb, dstb, zrow)
    return _gru(x, m1[:N], m2[:N], p3, W_ih, W_hh, b_ih, b_hh)
